# Initial kernel scaffold; baseline (speedup 1.0000x reference)
import jax
import jax.numpy as jnp
from jax.experimental import pallas as pl


def kernel(x, edge_index, edge_attr, node_irregularity, WQ, WK, WV, WE, Wout, bout, g1, b1, Wroot, Wneigh, bs, g2, b2, Wg1, bg1, Wg2, bg2, Wcls, bcls):
    raise NotImplementedError("write your pallas kernel here")


# speedup vs baseline: 15.0129x; 15.0129x over previous
import jax
import jax.numpy as jnp
import numpy as np
from jax import lax
from jax.experimental import pallas as pl
from jax.experimental.pallas import tpu as pltpu
from jax.experimental.pallas import tpu_sc as plsc

N = 10000
E = 320000
D = 128
NCLS = 40
GH = 64

NPAD = 10240
HP = NPAD // 2
HB = HP + 128
DUMMY = HP + 64
RPT = HP // 16
C = 128
NCHUNK = E // C
NWORK = 32
FULL_CHUNKS = NCHUNK // NWORK
PR = E // 8

_MESH = plsc.VectorSubcoreMesh(core_axis_name="c", subcore_axis_name="s")


def _lane_perm(v, idx16):
    dnums = lax.GatherDimensionNumbers(
        offset_dims=(), collapsed_slice_dims=(0,), start_index_map=(0,))
    return lax.gather(v, idx16[:, None], dnums, (1,),
                      mode=lax.GatherScatterMode.PROMISE_IN_BOUNDS)


def _zero_rows(ref, nrows, width):
    z = jnp.zeros((16,), jnp.float32)

    @pl.loop(0, nrows)
    def _(r):
        for c in range(width // 16):
            ref[r, 16 * c:16 * c + 16] = z


def _clamped_idx(dst_v, idx2_v, r):
    for k in range(C // 16):
        dv = dst_v[0, 16 * k:16 * k + 16]
        local = dv - r * HP
        ok = (local >= 0) & (local < HP)
        idx2_v[0, 16 * k:16 * k + 16] = jnp.where(ok, local, DUMMY)


def _init_half(src_ref, acc, sid, width):
    r0 = sid * RPT
    _zero_rows(src_ref, C, width)
    pltpu.sync_copy(src_ref, acc.at[pl.ds(r0, 128), :])
    pltpu.sync_copy(src_ref, acc.at[pl.ds(r0 + 128, 128), :])
    pltpu.sync_copy(src_ref.at[pl.ds(0, 64)], acc.at[pl.ds(r0 + 256, 64), :])


def _dump_half(acc, bounce_ref, out_slice_fn, sid):
    r0 = sid * RPT
    for off, ln in ((0, 128), (128, 128), (256, 64)):
        pltpu.sync_copy(acc.at[pl.ds(r0 + off, ln), :], bounce_ref.at[pl.ds(0, ln)])
        pltpu.sync_copy(bounce_ref.at[pl.ds(0, ln)], out_slice_fn(r0 + off, ln))


def _k1_body(ei, ea_hbm, aqk_hbm, aqe_hbm, bk_hbm, bv_hbm, x_hbm,
             p_out, msg_out, xg_out,
             src_v, dst_v, aqk_v, aqe_v, bk_v, bv_v, xg_v, ea_v, msg_v, p_v,
             sem0, sem1, sem2, sem3, sem4):
    cid = lax.axis_index("c")
    sid = lax.axis_index("s")
    wid = sid * 2 + cid

    iota = lax.iota(jnp.int32, 16)
    rot8 = iota ^ 8
    par = lax.shift_right_logical(iota, 3)

    @pl.loop(0, FULL_CHUNKS + 1)
    def _(g):
        chunk = g * NWORK + wid

        @pl.when(chunk < NCHUNK)
        def _():
            base = chunk * C
            pltpu.sync_copy(ei.at[0, pl.ds(base, C)], src_v.at[0])
            pltpu.sync_copy(ei.at[1, pl.ds(base, C)], dst_v.at[0])
            pltpu.sync_copy(ea_hbm.at[pl.ds(base, C), :], ea_v)
            cp0 = pltpu.async_copy(aqk_hbm.at[dst_v.at[0]], aqk_v, sem0)
            cp1 = pltpu.async_copy(aqe_hbm.at[dst_v.at[0]], aqe_v, sem1)
            cp2 = pltpu.async_copy(bk_hbm.at[src_v.at[0]], bk_v, sem2)
            cp3 = pltpu.async_copy(bv_hbm.at[src_v.at[0]], bv_v, sem3)
            cp4 = pltpu.async_copy(x_hbm.at[src_v.at[0]], xg_v, sem4)
            cp0.wait()
            cp1.wait()
            cp2.wait()
            cp3.wait()

            @pl.loop(0, C)
            def _(e):
                ea_row = ea_v[e, :]
                acc = aqk_v[e, 0:16] * bk_v[e, 0:16]
                for c in range(1, 8):
                    acc += aqk_v[e, 16 * c:16 * c + 16] * bk_v[e, 16 * c:16 * c + 16]
                for c in range(8):
                    eav = _lane_perm(ea_row, 2 * c + par)
                    acc += aqe_v[e, 16 * c:16 * c + 16] * eav
                score16 = acc + _lane_perm(acc, rot8)
                p16 = jnp.exp(score16)
                p_v[e // 8, pl.ds(16 * (e % 8), 16)] = p16
                for c in range(8):
                    msg_v[e, 16 * c:16 * c + 16] = p16 * bv_v[e, 16 * c:16 * c + 16]

            cp4.wait()
            pltpu.sync_copy(p_v, p_out.at[pl.ds(chunk * (C // 8), C // 8), :])
            pltpu.sync_copy(msg_v, msg_out.at[pl.ds(base, C), :])
            pltpu.sync_copy(xg_v, xg_out.at[pl.ds(base, C), :])


def _k2_body(ei, msg_hbm, agg_out,
             dst_v, idx2_v, msg_v, AGG):
    cid = lax.axis_index("c")
    sid = lax.axis_index("s")
    wid = sid * 2 + cid

    for r in range(2):
        _init_half(msg_v, AGG, sid, 128)
        plsc.subcore_barrier()

        @pl.loop(0, FULL_CHUNKS + 1)
        def _(g):
            chunk = g * NWORK + wid

            @pl.when(chunk < NCHUNK)
            def _():
                base = chunk * C
                pltpu.sync_copy(ei.at[1, pl.ds(base, C)], dst_v.at[0])
                pltpu.sync_copy(msg_hbm.at[pl.ds(base, C), :], msg_v)
                _clamped_idx(dst_v, idx2_v, r)
                pltpu.sync_copy(msg_v, AGG.at[idx2_v.at[0]], add=True)

        plsc.subcore_barrier()
        _dump_half(AGG, msg_v,
                   lambda a, n: agg_out.at[cid, pl.ds(r * HP + a, n), :], sid)
        plsc.subcore_barrier()


def _k2b_body(ei, p_hbm, s_out,
              dst_v, idx2_v, sw_v, p_v, S):
    cid = lax.axis_index("c")
    sid = lax.axis_index("s")
    wid = sid * 2 + cid

    iota = lax.iota(jnp.int32, 16)
    lt8 = iota < 8

    for r in range(2):
        _init_half(sw_v, S, sid, 128)
        plsc.subcore_barrier()

        @pl.loop(0, FULL_CHUNKS + 1)
        def _(g):
            chunk = g * NWORK + wid

            @pl.when(chunk < NCHUNK)
            def _():
                base = chunk * C
                pltpu.sync_copy(ei.at[1, pl.ds(base, C)], dst_v.at[0])
                pltpu.sync_copy(p_hbm.at[pl.ds(chunk * (C // 8), C // 8), :], p_v)
                _clamped_idx(dst_v, idx2_v, r)

                @pl.loop(0, C)
                def _(e):
                    p16 = p_v[e // 8, pl.ds(16 * (e % 8), 16)]
                    sw_v[e, 0:16] = jnp.where(lt8, p16, 1.0)

                pltpu.sync_copy(sw_v, S.at[idx2_v.at[0]], add=True)

        plsc.subcore_barrier()
        _dump_half(S, sw_v,
                   lambda a, n: s_out.at[cid, pl.ds(r * HP + a, n), :], sid)
        plsc.subcore_barrier()


def _k3a_body(ei, ea_hbm, p_hbm, beta_out,
              dst_v, idx2_v, ea_v, bu_v, p_v, BETA):
    cid = lax.axis_index("c")
    sid = lax.axis_index("s")
    wid = sid * 2 + cid

    iota = lax.iota(jnp.int32, 16)
    par = lax.shift_right_logical(iota, 3)

    for r in range(2):
        _init_half(bu_v, BETA, sid, 128)
        plsc.subcore_barrier()

        @pl.loop(0, FULL_CHUNKS + 1)
        def _(g):
            chunk = g * NWORK + wid

            @pl.when(chunk < NCHUNK)
            def _():
                base = chunk * C
                pltpu.sync_copy(ei.at[1, pl.ds(base, C)], dst_v.at[0])
                pltpu.sync_copy(ea_hbm.at[pl.ds(base, C), :], ea_v)
                pltpu.sync_copy(p_hbm.at[pl.ds(chunk * (C // 8), C // 8), :], p_v)
                _clamped_idx(dst_v, idx2_v, r)

                @pl.loop(0, C)
                def _(e):
                    ea_row = ea_v[e, :]
                    p16 = p_v[e // 8, pl.ds(16 * (e % 8), 16)]
                    for c in range(8):
                        eav = _lane_perm(ea_row, 2 * c + par)
                        bu_v[e, 16 * c:16 * c + 16] = p16 * eav

                pltpu.sync_copy(bu_v, BETA.at[idx2_v.at[0]], add=True)

        plsc.subcore_barrier()
        _dump_half(BETA, bu_v,
                   lambda a, n: beta_out.at[cid, pl.ds(r * HP + a, n), :], sid)
        plsc.subcore_barrier()


def _k3b_body(ei, xg_hbm, nb_out,
              dst_v, idx2_v, x_v, NB):
    cid = lax.axis_index("c")
    sid = lax.axis_index("s")
    wid = sid * 2 + cid

    for r in range(2):
        _init_half(x_v, NB, sid, 128)
        plsc.subcore_barrier()

        @pl.loop(0, FULL_CHUNKS + 1)
        def _(g):
            chunk = g * NWORK + wid

            @pl.when(chunk < NCHUNK)
            def _():
                base = chunk * C
                pltpu.sync_copy(ei.at[1, pl.ds(base, C)], dst_v.at[0])
                pltpu.sync_copy(xg_hbm.at[pl.ds(base, C), :], x_v)
                _clamped_idx(dst_v, idx2_v, r)
                pltpu.sync_copy(x_v, NB.at[idx2_v.at[0]], add=True)

        plsc.subcore_barrier()
        _dump_half(NB, x_v,
                   lambda a, n: nb_out.at[cid, pl.ds(r * HP + a, n), :], sid)
        plsc.subcore_barrier()


def _pre_body(x_ref, wqk_ref, wqe_ref, wk_ref, wv_ref,
              aqk_ref, aqe_ref, bk_ref, bv_ref):
    xb = x_ref[...]
    aqk_ref[...] = jnp.dot(xb, wqk_ref[...], preferred_element_type=jnp.float32)
    aqe_ref[...] = jnp.dot(xb, wqe_ref[...], preferred_element_type=jnp.float32)
    bk_ref[...] = jnp.dot(xb, wk_ref[...], preferred_element_type=jnp.float32)
    bv_ref[...] = jnp.dot(xb, wv_ref[...], preferred_element_type=jnp.float32)


def _post1_body(x_ref, agg_ref, s_ref, beta_ref, nb_ref,
                mtt_ref, woutp_ref, bout_ref, wroot_ref, wneigh_ref, bs_ref,
                z1_ref, mac_ref, stats_ref, acc_ref):
    i = pl.program_id(0)
    xb = x_ref[...]
    ssum = s_ref[0] + s_ref[1]
    s8t = jnp.tile(ssum[:, :8], (1, 16)) + 1e-16
    aggc = agg_ref[0] + agg_ref[1]
    betac = beta_ref[0] + beta_ref[1]
    agg_t = (aggc + jnp.dot(betac, mtt_ref[...], preferred_element_type=jnp.float32)) / s8t
    z1 = jnp.dot(agg_t, woutp_ref[...], preferred_element_type=jnp.float32) + bout_ref[...] + xb
    deg = jnp.maximum(ssum[:, 8:9], 1.0)
    nb = (nb_ref[0] + nb_ref[1]) / deg
    mac = (jnp.dot(xb, wroot_ref[...], preferred_element_type=jnp.float32)
           + jnp.dot(nb, wneigh_ref[...], preferred_element_type=jnp.float32)
           + bs_ref[...])
    z1_ref[...] = z1
    mac_ref[...] = mac

    @pl.when(i == 0)
    def _():
        acc_ref[...] = jnp.zeros_like(acc_ref)

    acc_ref[0:1, :] += jnp.sum(z1, axis=0, keepdims=True)
    acc_ref[1:2, :] += jnp.sum(z1 * z1, axis=0, keepdims=True)
    acc_ref[2:3, :] += jnp.sum(mac, axis=0, keepdims=True)
    acc_ref[3:4, :] += jnp.sum(mac * mac, axis=0, keepdims=True)
    stats_ref[...] = acc_ref[...]


def _post2_body(z1_ref, mac_ref, stats_ref, x_ref, irr_ref,
                g1_ref, b1_ref, g2_ref, b2_ref,
                wg1a_ref, wg1b_ref, wg1c_ref, bg1_ref, wg2_ref, bg2_ref,
                wcls_ref, bcls_ref, out_ref):
    stats = stats_ref[...]
    mu1 = stats[0:1, :] / N
    var1 = stats[1:2, :] / N - mu1 * mu1
    mu2 = stats[2:3, :] / N
    var2 = stats[3:4, :] / N - mu2 * mu2
    z1 = z1_ref[...]
    mac_pre = mac_ref[...]
    xb = x_ref[...]
    micro = jax.nn.gelu((z1 - mu1) / jnp.sqrt(var1 + 1e-5) * g1_ref[...] + b1_ref[...])
    macro = jax.nn.gelu((mac_pre - mu2) / jnp.sqrt(var2 + 1e-5) * g2_ref[...] + b2_ref[...]) + xb
    gh = jax.nn.silu(
        jnp.dot(macro, wg1a_ref[...], preferred_element_type=jnp.float32)
        + jnp.dot(micro, wg1b_ref[...], preferred_element_type=jnp.float32)
        + irr_ref[...] * wg1c_ref[...]
        + bg1_ref[...])
    a = jax.nn.sigmoid(jnp.dot(gh, wg2_ref[...], preferred_element_type=jnp.float32) + bg2_ref[...])
    fused = a * macro + (1.0 - a) * micro
    out_ref[...] = jnp.dot(fused, wcls_ref[...], preferred_element_type=jnp.float32) + bcls_ref[...]


BN = 1000
NB_TC = N // BN


def kernel(x, edge_index, edge_attr, node_irregularity, WQ, WK, WV, WE, Wout,
           bout, g1, b1, Wroot, Wneigh, bs, g2, b2, Wg1, bg1, Wg2, bg2, Wcls, bcls):
    f32 = jnp.float32
    d_idx = np.arange(16)
    h_idx = np.arange(8)
    perm = (h_idx[None, :] * 16 + d_idx[:, None]).reshape(-1)
    WQP = WQ[:, perm]
    W3 = WE.reshape(16, 8, 16).transpose(2, 1, 0)
    MT = (W3[:, :, :, None] * jnp.eye(8, dtype=f32)[None, :, None, :]).reshape(128, 128)

    aqk, aqe, bk, bv = pl.pallas_call(
        _pre_body,
        grid=(NB_TC,),
        in_specs=[pl.BlockSpec((BN, 128), lambda i: (i, 0))] +
                 [pl.BlockSpec((128, 128), lambda i: (0, 0))] * 4,
        out_specs=[pl.BlockSpec((BN, 128), lambda i: (i, 0))] * 4,
        out_shape=[jax.ShapeDtypeStruct((N, 128), f32)] * 4,
    )(x, WQP / 4.0, (WQP @ MT) / 4.0, WK[:, perm], WV[:, perm])

    p_edge, msg_e, xg_e = pl.kernel(
        _k1_body,
        out_type=[jax.ShapeDtypeStruct((PR, 128), f32),
                  jax.ShapeDtypeStruct((E, 128), f32),
                  jax.ShapeDtypeStruct((E, 128), f32)],
        mesh=_MESH,
        scratch_types=[
            pltpu.VMEM((1, C), jnp.int32),
            pltpu.VMEM((1, C), jnp.int32),
            pltpu.VMEM((C, 128), f32),
            pltpu.VMEM((C, 128), f32),
            pltpu.VMEM((C, 128), f32),
            pltpu.VMEM((C, 128), f32),
            pltpu.VMEM((C, 128), f32),
            pltpu.VMEM((C, 16), f32),
            pltpu.VMEM((C, 128), f32),
            pltpu.VMEM((C // 8, 128), f32),
            pltpu.SemaphoreType.DMA,
            pltpu.SemaphoreType.DMA,
            pltpu.SemaphoreType.DMA,
            pltpu.SemaphoreType.DMA,
            pltpu.SemaphoreType.DMA,
        ],
    )(edge_index, edge_attr, aqk, aqe, bk, bv, x)

    agg_p = pl.kernel(
        _k2_body,
        out_type=jax.ShapeDtypeStruct((2, NPAD, 128), f32),
        mesh=_MESH,
        scratch_types=[
            pltpu.VMEM((1, C), jnp.int32),
            pltpu.VMEM((1, C), jnp.int32),
            pltpu.VMEM((C, 128), f32),
            pltpu.VMEM_SHARED((HB, 128), f32),
        ],
    )(edge_index, msg_e)

    s_p = pl.kernel(
        _k2b_body,
        out_type=jax.ShapeDtypeStruct((2, NPAD, 128), f32),
        mesh=_MESH,
        scratch_types=[
            pltpu.VMEM((1, C), jnp.int32),
            pltpu.VMEM((1, C), jnp.int32),
            pltpu.VMEM((C, 128), f32),
            pltpu.VMEM((C // 8, 128), f32),
            pltpu.VMEM_SHARED((HB, 128), f32),
        ],
    )(edge_index, p_edge)

    beta_p = pl.kernel(
        _k3a_body,
        out_type=jax.ShapeDtypeStruct((2, NPAD, 128), f32),
        mesh=_MESH,
        scratch_types=[
            pltpu.VMEM((1, C), jnp.int32),
            pltpu.VMEM((1, C), jnp.int32),
            pltpu.VMEM((C, 16), f32),
            pltpu.VMEM((C, 128), f32),
            pltpu.VMEM((C // 8, 128), f32),
            pltpu.VMEM_SHARED((HB, 128), f32),
        ],
    )(edge_index, edge_attr, p_edge)

    nb_p = pl.kernel(
        _k3b_body,
        out_type=jax.ShapeDtypeStruct((2, NPAD, 128), f32),
        mesh=_MESH,
        scratch_types=[
            pltpu.VMEM((1, C), jnp.int32),
            pltpu.VMEM((1, C), jnp.int32),
            pltpu.VMEM((C, 128), f32),
            pltpu.VMEM_SHARED((HB, 128), f32),
        ],
    )(edge_index, xg_e)

    z1, mac, stats = pl.pallas_call(
        _post1_body,
        grid=(NB_TC,),
        in_specs=[pl.BlockSpec((BN, 128), lambda i: (i, 0)),
                  pl.BlockSpec((2, BN, 128), lambda i: (0, i, 0)),
                  pl.BlockSpec((2, BN, 128), lambda i: (0, i, 0)),
                  pl.BlockSpec((2, BN, 128), lambda i: (0, i, 0)),
                  pl.BlockSpec((2, BN, 128), lambda i: (0, i, 0)),
                  pl.BlockSpec((128, 128), lambda i: (0, 0)),
                  pl.BlockSpec((128, 128), lambda i: (0, 0)),
                  pl.BlockSpec((1, 128), lambda i: (0, 0)),
                  pl.BlockSpec((128, 128), lambda i: (0, 0)),
                  pl.BlockSpec((128, 128), lambda i: (0, 0)),
                  pl.BlockSpec((1, 128), lambda i: (0, 0))],
        out_specs=[pl.BlockSpec((BN, 128), lambda i: (i, 0)),
                   pl.BlockSpec((BN, 128), lambda i: (i, 0)),
                   pl.BlockSpec((8, 128), lambda i: (0, 0))],
        out_shape=[jax.ShapeDtypeStruct((N, 128), f32),
                   jax.ShapeDtypeStruct((N, 128), f32),
                   jax.ShapeDtypeStruct((8, 128), f32)],
        scratch_shapes=[pltpu.VMEM((8, 128), f32)],
    )(x, agg_p, s_p, beta_p, nb_p,
      MT.T, Wout[perm, :], bout.reshape(1, 128), Wroot, Wneigh, bs.reshape(1, 128))

    logits = pl.pallas_call(
        _post2_body,
        grid=(NB_TC,),
        in_specs=[pl.BlockSpec((BN, 128), lambda i: (i, 0)),
                  pl.BlockSpec((BN, 128), lambda i: (i, 0)),
                  pl.BlockSpec((8, 128), lambda i: (0, 0)),
                  pl.BlockSpec((BN, 128), lambda i: (i, 0)),
                  pl.BlockSpec((BN, 1), lambda i: (i, 0)),
                  pl.BlockSpec((1, 128), lambda i: (0, 0)),
                  pl.BlockSpec((1, 128), lambda i: (0, 0)),
                  pl.BlockSpec((1, 128), lambda i: (0, 0)),
                  pl.BlockSpec((1, 128), lambda i: (0, 0)),
                  pl.BlockSpec((128, GH), lambda i: (0, 0)),
                  pl.BlockSpec((128, GH), lambda i: (0, 0)),
                  pl.BlockSpec((1, GH), lambda i: (0, 0)),
                  pl.BlockSpec((1, GH), lambda i: (0, 0)),
                  pl.BlockSpec((GH, 1), lambda i: (0, 0)),
                  pl.BlockSpec((1, 1), lambda i: (0, 0)),
                  pl.BlockSpec((128, NCLS), lambda i: (0, 0)),
                  pl.BlockSpec((1, NCLS), lambda i: (0, 0))],
        out_specs=pl.BlockSpec((BN, NCLS), lambda i: (i, 0)),
        out_shape=jax.ShapeDtypeStruct((N, NCLS), f32),
    )(z1, mac, stats, x, node_irregularity.reshape(N, 1),
      g1.reshape(1, 128), b1.reshape(1, 128), g2.reshape(1, 128), b2.reshape(1, 128),
      Wg1[:128], Wg1[128:256], Wg1[256:257], bg1.reshape(1, GH),
      Wg2, bg2.reshape(1, 1), Wcls, bcls.reshape(1, NCLS))

    return logits
